# baseline (speedup 1.0000x reference)
import functools

import numpy as np
import jax
import jax.numpy as jnp
from jax.experimental import pallas as pl
from jax.experimental.pallas import tpu as pltpu

_TOP_K = 16
_L = 8192
_N1 = 64
_N2 = 128
_K2 = _N2 // 2 + 1
_CB = 128


def _build_consts():
    L, N1, N2, K2, CB = _L, _N1, _N2, _K2, _CB
    n1 = np.arange(N1)
    n2 = np.arange(N2)
    k1 = np.arange(N1)
    k2 = np.arange(K2)
    p = np.arange(N2)
    f32 = np.float32

    D1 = np.exp(-2j * np.pi * np.outer(k1, n1) / N1)
    T1 = np.exp(-2j * np.pi * np.outer(k1, n2) / L)
    T1b = np.broadcast_to(T1[:, :, None], (N1, N2, CB)).copy()
    D2 = np.exp(-2j * np.pi * np.outer(k2, n2) / N2)

    E2 = np.exp(2j * np.pi * np.outer(p, k2) / N2)
    T2 = np.exp(2j * np.pi * np.outer(k1, p) / L)
    T2b = np.broadcast_to(np.transpose(T2)[:, :, None], (N2, N1, CB)).copy()
    E1 = np.exp(2j * np.pi * np.outer(np.arange(N1), k1) / N1)

    fwd = (
        f32(D1.real), f32(D1.imag),
        f32(T1b.real), f32(T1b.imag),
        f32(D2.real), f32(D2.imag),
    )
    inv = (
        f32(E2.real), f32(E2.imag),
        f32(T2b.real), f32(T2b.imag),
        f32(E1.real), f32(E1.imag),
    )
    return fwd, inv


_DOT = functools.partial(
    jax.lax.dot_general,
    precision=jax.lax.Precision.HIGHEST,
    preferred_element_type=jnp.float32,
)
_DN_LEAD = (((1,), (0,)), ((), ()))
_DN_MID = (((1,), (1,)), ((), ()))


def _fwd_body(x_ref, d1r_ref, d1i_ref, t1r_ref, t1i_ref, d2r_ref, d2i_ref,
              yr_ref, yi_ref):
    N1, N2, K2, L = _N1, _N2, _K2, _L
    CB = x_ref.shape[2]
    x3 = x_ref[0].reshape(N1, N2, CB)

    ar = _DOT(d1r_ref[...], x3, dimension_numbers=_DN_LEAD)
    ai = _DOT(d1i_ref[...], x3, dimension_numbers=_DN_LEAD)

    t1r = t1r_ref[...]
    t1i = t1i_ref[...]
    br = ar * t1r - ai * t1i
    bi = ar * t1i + ai * t1r

    d2r = d2r_ref[...]
    d2i = d2i_ref[...]
    xr = _DOT(d2r, br, dimension_numbers=_DN_MID) - _DOT(
        d2i, bi, dimension_numbers=_DN_MID)
    xi = _DOT(d2r, bi, dimension_numbers=_DN_MID) + _DOT(
        d2i, br, dimension_numbers=_DN_MID)

    i2 = jax.lax.broadcasted_iota(jnp.int32, (K2, N1, CB), 0)
    i1 = jax.lax.broadcasted_iota(jnp.int32, (K2, N1, CB), 1)
    kfull = i1 + N1 * i2
    valid = kfull <= (L // 2)

    mag = jnp.sqrt(xr * xr + xi * xi)
    work = jnp.where(valid, mag, -0.5)
    big = jnp.int32(L)

    def step(_, w):
        vm = jnp.max(w, axis=1, keepdims=True)
        v = jnp.max(vm, axis=0, keepdims=True)
        cand = jnp.where(w == v, kfull, big)
        rm = jnp.min(cand, axis=1, keepdims=True)
        r = jnp.min(rm, axis=0, keepdims=True)
        return jnp.where(kfull == r, -1.0, w)

    work = jax.lax.fori_loop(0, _TOP_K, step, work)
    keep = work < -0.9

    wgt = jnp.where(
        (kfull == 0) | (kfull == L // 2), 1.0 / L, 2.0 / L
    ).astype(jnp.float32)
    wgt = jnp.where(keep, wgt, 0.0)
    yr_ref[0] = jnp.transpose(xr * wgt, (1, 0, 2))
    yi_ref[0] = jnp.transpose(xi * wgt, (1, 0, 2))


def _inv_body(yr_ref, yi_ref, e2r_ref, e2i_ref, t2r_ref, t2i_ref,
              e1r_ref, e1i_ref, o_ref):
    N1, N2 = _N1, _N2
    CB = yr_ref.shape[3]
    yr = yr_ref[0]
    yi = yi_ref[0]

    e2r = e2r_ref[...]
    e2i = e2i_ref[...]
    qr = _DOT(e2r, yr, dimension_numbers=_DN_MID) - _DOT(
        e2i, yi, dimension_numbers=_DN_MID)
    qi = _DOT(e2r, yi, dimension_numbers=_DN_MID) + _DOT(
        e2i, yr, dimension_numbers=_DN_MID)

    t2r = t2r_ref[...]
    t2i = t2i_ref[...]
    gr = qr * t2r - qi * t2i
    gi = qr * t2i + qi * t2r

    out = _DOT(e1r_ref[...], gr, dimension_numbers=_DN_MID) - _DOT(
        e1i_ref[...], gi, dimension_numbers=_DN_MID)
    o_ref[0] = out.reshape(N1 * N2, CB)


@jax.jit
def kernel(x):
    B, L, C = x.shape
    fwd_c, inv_c = _build_consts()
    grid = (B, C // _CB)
    xspec = pl.BlockSpec((1, L, _CB), lambda b, c: (b, 0, c))
    yspec = pl.BlockSpec((1, _N1, _K2, _CB), lambda b, c: (b, 0, 0, c))

    def cspec(shape):
        nd = len(shape)
        return pl.BlockSpec(shape, lambda b, c: (0,) * nd)

    yshape = jax.ShapeDtypeStruct((B, _N1, _K2, C), jnp.float32)
    yr, yi = pl.pallas_call(
        _fwd_body,
        grid=grid,
        in_specs=[xspec] + [cspec(a.shape) for a in fwd_c],
        out_specs=[yspec, yspec],
        out_shape=[yshape, yshape],
        compiler_params=pltpu.CompilerParams(
            vmem_limit_bytes=63 * 1024 * 1024,
        ),
    )(x, *fwd_c)

    return pl.pallas_call(
        _inv_body,
        grid=grid,
        in_specs=[yspec, yspec] + [cspec(a.shape) for a in inv_c],
        out_specs=xspec,
        out_shape=jax.ShapeDtypeStruct((B, L, C), jnp.float32),
        compiler_params=pltpu.CompilerParams(
            vmem_limit_bytes=63 * 1024 * 1024,
        ),
    )(yr, yi, *inv_c)


# speedup vs baseline: 1.2102x; 1.2102x over previous
import functools

import numpy as np
import jax
import jax.numpy as jnp
from jax.experimental import pallas as pl
from jax.experimental.pallas import tpu as pltpu

_TOP_K = 16
_L = 8192
_N1 = 64
_N2 = 128
_K2 = _N2 // 2 + 1
_CB = 128


def _build_consts():
    L, N1, N2, K2, CB = _L, _N1, _N2, _K2, _CB
    n1 = np.arange(N1)
    n2 = np.arange(N2)
    k1 = np.arange(N1)
    k2 = np.arange(K2)
    p = np.arange(N2)
    f32 = np.float32

    D1 = np.exp(-2j * np.pi * np.outer(k1, n1) / N1)
    T1 = np.exp(-2j * np.pi * np.outer(k1, n2) / L)
    T1b = np.broadcast_to(T1[:, :, None], (N1, N2, CB)).copy()
    D2 = np.exp(-2j * np.pi * np.outer(k2, n2) / N2)

    E2 = np.exp(2j * np.pi * np.outer(p, k2) / N2)
    T2 = np.exp(2j * np.pi * np.outer(k1, p) / L)
    T2b = np.broadcast_to(np.transpose(T2)[:, :, None], (N2, N1, CB)).copy()
    E1 = np.exp(2j * np.pi * np.outer(np.arange(N1), k1) / N1)

    fwd = (
        f32(D1.real), f32(D1.imag),
        f32(T1b.real), f32(T1b.imag),
        f32(D2.real), f32(D2.imag),
    )
    inv = (
        f32(E2.real), f32(E2.imag),
        f32(T2b.real), f32(T2b.imag),
        f32(E1.real), f32(E1.imag),
    )
    return fwd, inv


_DOT = functools.partial(
    jax.lax.dot_general,
    precision=jax.lax.Precision.HIGHEST,
    preferred_element_type=jnp.float32,
)
_DOTF = functools.partial(
    jax.lax.dot_general,
    precision=jax.lax.Precision.DEFAULT,
    preferred_element_type=jnp.float32,
)
_DN_LEAD = (((1,), (0,)), ((), ()))
_DN_MID = (((1,), (1,)), ((), ()))


def _fwd_body(x_ref, d1r_ref, d1i_ref, t1r_ref, t1i_ref, d2r_ref, d2i_ref,
              yr_ref, yi_ref):
    N1, N2, K2, L = _N1, _N2, _K2, _L
    CB = x_ref.shape[2]
    x3 = x_ref[0].reshape(N1, N2, CB)

    ar = _DOT(d1r_ref[...], x3, dimension_numbers=_DN_LEAD)
    ai = _DOT(d1i_ref[...], x3, dimension_numbers=_DN_LEAD)

    t1r = t1r_ref[...]
    t1i = t1i_ref[...]
    br = ar * t1r - ai * t1i
    bi = ar * t1i + ai * t1r

    d2r = d2r_ref[...]
    d2i = d2i_ref[...]
    xr = _DOT(d2r, br, dimension_numbers=_DN_MID) - _DOT(
        d2i, bi, dimension_numbers=_DN_MID)
    xi = _DOT(d2r, bi, dimension_numbers=_DN_MID) + _DOT(
        d2i, br, dimension_numbers=_DN_MID)

    i2 = jax.lax.broadcasted_iota(jnp.int32, (K2, N1, CB), 0)
    i1 = jax.lax.broadcasted_iota(jnp.int32, (K2, N1, CB), 1)
    kfull = i1 + N1 * i2
    valid = kfull <= (L // 2)

    mag = jnp.sqrt(xr * xr + xi * xi)
    work = jnp.where(valid, mag, -0.5)
    big = jnp.int32(L)

    def step(_, w):
        vm = jnp.max(w, axis=1, keepdims=True)
        v = jnp.max(vm, axis=0, keepdims=True)
        cand = jnp.where(w == v, kfull, big)
        rm = jnp.min(cand, axis=1, keepdims=True)
        r = jnp.min(rm, axis=0, keepdims=True)
        return jnp.where(kfull == r, -1.0, w)

    work = jax.lax.fori_loop(0, _TOP_K, step, work)
    keep = work < -0.9

    wgt = jnp.where(
        (kfull == 0) | (kfull == L // 2), 1.0 / L, 2.0 / L
    ).astype(jnp.float32)
    wgt = jnp.where(keep, wgt, 0.0)
    yr_ref[0] = jnp.transpose(xr * wgt, (1, 0, 2))
    yi_ref[0] = jnp.transpose(xi * wgt, (1, 0, 2))


def _inv_body(yr_ref, yi_ref, e2r_ref, e2i_ref, t2r_ref, t2i_ref,
              e1r_ref, e1i_ref, o_ref):
    N1, N2 = _N1, _N2
    CB = yr_ref.shape[3]
    yr = yr_ref[0]
    yi = yi_ref[0]

    e2r = e2r_ref[...]
    e2i = e2i_ref[...]
    qr = _DOTF(e2r, yr, dimension_numbers=_DN_MID) - _DOTF(
        e2i, yi, dimension_numbers=_DN_MID)
    qi = _DOTF(e2r, yi, dimension_numbers=_DN_MID) + _DOTF(
        e2i, yr, dimension_numbers=_DN_MID)

    t2r = t2r_ref[...]
    t2i = t2i_ref[...]
    gr = qr * t2r - qi * t2i
    gi = qr * t2i + qi * t2r

    out = _DOTF(e1r_ref[...], gr, dimension_numbers=_DN_MID) - _DOTF(
        e1i_ref[...], gi, dimension_numbers=_DN_MID)
    o_ref[0] = out.reshape(N1 * N2, CB)


@jax.jit
def kernel(x):
    B, L, C = x.shape
    fwd_c, inv_c = _build_consts()
    grid = (B, C // _CB)
    xspec = pl.BlockSpec((1, L, _CB), lambda b, c: (b, 0, c))
    yspec = pl.BlockSpec((1, _N1, _K2, _CB), lambda b, c: (b, 0, 0, c))

    def cspec(shape):
        nd = len(shape)
        return pl.BlockSpec(shape, lambda b, c: (0,) * nd)

    yshape = jax.ShapeDtypeStruct((B, _N1, _K2, C), jnp.float32)
    yr, yi = pl.pallas_call(
        _fwd_body,
        grid=grid,
        in_specs=[xspec] + [cspec(a.shape) for a in fwd_c],
        out_specs=[yspec, yspec],
        out_shape=[yshape, yshape],
        compiler_params=pltpu.CompilerParams(
            vmem_limit_bytes=63 * 1024 * 1024,
        ),
    )(x, *fwd_c)

    return pl.pallas_call(
        _inv_body,
        grid=grid,
        in_specs=[yspec, yspec] + [cspec(a.shape) for a in inv_c],
        out_specs=xspec,
        out_shape=jax.ShapeDtypeStruct((B, L, C), jnp.float32),
        compiler_params=pltpu.CompilerParams(
            vmem_limit_bytes=63 * 1024 * 1024,
        ),
    )(yr, yi, *inv_c)


# speedup vs baseline: 1.2477x; 1.0310x over previous
import functools

import numpy as np
import jax
import jax.numpy as jnp
from jax.experimental import pallas as pl
from jax.experimental.pallas import tpu as pltpu

_TOP_K = 16
_L = 8192
_N1 = 64
_N2 = 128
_K2 = _N2 // 2 + 1
_CB = 128


def _build_consts():
    L, N1, N2, K2, CB = _L, _N1, _N2, _K2, _CB
    n1 = np.arange(N1)
    n2 = np.arange(N2)
    k1 = np.arange(N1)
    k2 = np.arange(K2)
    p = np.arange(N2)
    f32 = np.float32

    D1 = np.exp(-2j * np.pi * np.outer(k1, n1) / N1)
    T1 = np.exp(-2j * np.pi * np.outer(k1, n2) / L)
    T1b = np.broadcast_to(T1[:, :, None], (N1, N2, CB)).copy()
    D2 = np.exp(-2j * np.pi * np.outer(k2, n2) / N2)

    E2 = np.exp(2j * np.pi * np.outer(p, k2) / N2)
    T2 = np.exp(2j * np.pi * np.outer(k1, p) / L)
    T2b = np.broadcast_to(np.transpose(T2)[:, :, None], (N2, N1, CB)).copy()
    E1 = np.exp(2j * np.pi * np.outer(np.arange(N1), k1) / N1)

    fwd = (
        f32(D1.real), f32(D1.imag),
        f32(T1b.real), f32(T1b.imag),
        f32(D2.real), f32(D2.imag),
    )
    inv = (
        f32(E2.real), f32(E2.imag),
        f32(T2b.real), f32(T2b.imag),
        f32(E1.real), f32(E1.imag),
    )
    return fwd, inv


_DOT = functools.partial(
    jax.lax.dot_general,
    precision=jax.lax.Precision.HIGHEST,
    preferred_element_type=jnp.float32,
)
_DOTF = functools.partial(
    jax.lax.dot_general,
    precision=jax.lax.Precision.DEFAULT,
    preferred_element_type=jnp.float32,
)
_DN_LEAD = (((1,), (0,)), ((), ()))
_DN_MID = (((1,), (1,)), ((), ()))


def _fused_body(x_ref, d1r_ref, d1i_ref, t1r_ref, t1i_ref, d2r_ref, d2i_ref,
                e2r_ref, e2i_ref, t2r_ref, t2i_ref, e1r_ref, e1i_ref, o_ref):
    N1, N2, K2, L = _N1, _N2, _K2, _L
    CB = x_ref.shape[2]
    x3 = x_ref[0].reshape(N1, N2, CB)

    ar = _DOT(d1r_ref[...], x3, dimension_numbers=_DN_LEAD)
    ai = _DOT(d1i_ref[...], x3, dimension_numbers=_DN_LEAD)

    t1r = t1r_ref[...]
    t1i = t1i_ref[...]
    br = ar * t1r - ai * t1i
    bi = ar * t1i + ai * t1r

    d2r = d2r_ref[...]
    d2i = d2i_ref[...]
    xr = _DOT(d2r, br, dimension_numbers=_DN_MID) - _DOT(
        d2i, bi, dimension_numbers=_DN_MID)
    xi = _DOT(d2r, bi, dimension_numbers=_DN_MID) + _DOT(
        d2i, br, dimension_numbers=_DN_MID)

    i2 = jax.lax.broadcasted_iota(jnp.int32, (K2, N1, CB), 0)
    i1 = jax.lax.broadcasted_iota(jnp.int32, (K2, N1, CB), 1)
    kfull = i1 + N1 * i2
    valid = kfull <= (L // 2)

    mag = jnp.sqrt(xr * xr + xi * xi)
    work = jnp.where(valid, mag, -0.5)
    big = jnp.int32(L)

    def step(_, w):
        vm = jnp.max(w, axis=1, keepdims=True)
        v = jnp.max(vm, axis=0, keepdims=True)
        cand = jnp.where(w == v, kfull, big)
        rm = jnp.min(cand, axis=1, keepdims=True)
        r = jnp.min(rm, axis=0, keepdims=True)
        return jnp.where(kfull == r, -1.0, w)

    work = jax.lax.fori_loop(0, _TOP_K, step, work)
    keep = work < -0.9

    wgt = jnp.where(
        (kfull == 0) | (kfull == L // 2), 1.0 / L, 2.0 / L
    ).astype(jnp.float32)
    wgt = jnp.where(keep, wgt, 0.0)
    yr = jnp.transpose(xr * wgt, (1, 0, 2))
    yi = jnp.transpose(xi * wgt, (1, 0, 2))

    e2r = e2r_ref[...]
    e2i = e2i_ref[...]
    qr = _DOTF(e2r, yr, dimension_numbers=_DN_MID) - _DOTF(
        e2i, yi, dimension_numbers=_DN_MID)
    qi = _DOTF(e2r, yi, dimension_numbers=_DN_MID) + _DOTF(
        e2i, yr, dimension_numbers=_DN_MID)

    t2r = t2r_ref[...]
    t2i = t2i_ref[...]
    gr = qr * t2r - qi * t2i
    gi = qr * t2i + qi * t2r

    out = _DOTF(e1r_ref[...], gr, dimension_numbers=_DN_MID) - _DOTF(
        e1i_ref[...], gi, dimension_numbers=_DN_MID)
    o_ref[0] = out.reshape(N1 * N2, CB)


@jax.jit
def kernel(x):
    B, L, C = x.shape
    fwd_c, inv_c = _build_consts()
    grid = (B, C // _CB)
    xspec = pl.BlockSpec((1, L, _CB), lambda b, c: (b, 0, c))

    def cspec(shape):
        nd = len(shape)
        return pl.BlockSpec(shape, lambda b, c: (0,) * nd)

    carrs = fwd_c + inv_c
    return pl.pallas_call(
        _fused_body,
        grid=grid,
        in_specs=[xspec] + [cspec(a.shape) for a in carrs],
        out_specs=xspec,
        out_shape=jax.ShapeDtypeStruct((B, L, C), jnp.float32),
        compiler_params=pltpu.CompilerParams(
            vmem_limit_bytes=63 * 1024 * 1024,
        ),
    )(x, *carrs)
